# baseline (speedup 1.0000x reference)
import functools

import jax
import jax.numpy as jnp
from jax import lax
from jax.experimental import pallas as pl
from jax.experimental.pallas import tpu as pltpu
from jax.experimental.pallas import tpu_sc as plsc

_VOCAB = 100000
_EMBED_DIM = 64
_N_TOKENS = 10
_BATCH = 32
_SEQ = 2048

_CHUNK = 128
_N_CHUNKS = _SEQ // _CHUNK
_GROUP = 2
_N_GROUPS = _N_CHUNKS // _GROUP


def _build_sc_kernel():
    mesh = plsc.VectorSubcoreMesh(core_axis_name="c", subcore_axis_name="s")

    @functools.partial(
        pl.kernel,
        mesh=mesh,
        compiler_params=pltpu.CompilerParams(use_tc_tiling_on_sc=False),
        out_type=jax.ShapeDtypeStruct((_BATCH * _SEQ, _EMBED_DIM), jnp.float32),
        scratch_types=[
            pltpu.VMEM((_N_CHUNKS, _CHUNK), jnp.int32),
            pltpu.VMEM((_GROUP * _CHUNK, _EMBED_DIM), jnp.float32),
            pltpu.VMEM((_GROUP * _CHUNK, _EMBED_DIM), jnp.float32),
            pltpu.VMEM((_N_TOKENS, _EMBED_DIM), jnp.float32),
            pltpu.SemaphoreType.DMA,
            pltpu.SemaphoreType.DMA,
            pltpu.SemaphoreType.DMA,
            pltpu.SemaphoreType.DMA,
        ],
    )
    def k(tok_hbm, table_hbm, learned_hbm, out_hbm,
          idx_v, rows0, rows1, learned_v, gsem0, gsem1, wsem0, wsem1):
        wid = lax.axis_index("s") * 2 + lax.axis_index("c")
        base = wid * _SEQ

        pltpu.sync_copy(tok_hbm.at[wid], idx_v)
        pltpu.sync_copy(learned_hbm, learned_v)

        bufs = (rows0, rows1)
        gsems = (gsem0, gsem1)
        wsems = (wsem0, wsem1)
        gcopies = [[None] * _GROUP, [None] * _GROUP]
        wcopy = [None, None]

        def fire(g, p):
            for t in range(_GROUP):
                gcopies[p][t] = pltpu.async_copy(
                    table_hbm.at[idx_v.at[g * _GROUP + t]],
                    bufs[p].at[pl.ds(t * _CHUNK, _CHUNK)],
                    gsems[p],
                )

        fire(0, 0)
        for g in range(_N_GROUPS):
            p = g % 2
            for t in range(_GROUP):
                gcopies[p][t].wait()
            if g + 1 < _N_GROUPS:
                if wcopy[1 - p] is not None:
                    wcopy[1 - p].wait()
                    wcopy[1 - p] = None
                fire(g + 1, 1 - p)
            if g == 0:
                for r in range(_N_TOKENS):
                    for c in range(_EMBED_DIM // 16):
                        bufs[p][r, pl.ds(c * 16, 16)] = learned_v[r, pl.ds(c * 16, 16)]
            wcopy[p] = pltpu.async_copy(
                bufs[p],
                out_hbm.at[pl.ds(base + g * _GROUP * _CHUNK, _GROUP * _CHUNK)],
                wsems[p],
            )
        for p in (0, 1):
            if wcopy[p] is not None:
                wcopy[p].wait()

    return k


_sc_kernel = _build_sc_kernel()


@jax.jit
def kernel(tokens, wte_weight, learned_embedding):
    tok = tokens.astype(jnp.int32).reshape(_BATCH, _N_CHUNKS, _CHUNK)
    out = _sc_kernel(tok, wte_weight, learned_embedding)
    return out.reshape(_BATCH, _SEQ, _EMBED_DIM)


# speedup vs baseline: 1.0148x; 1.0148x over previous
import functools

import jax
import jax.numpy as jnp
from jax import lax
from jax.experimental import pallas as pl
from jax.experimental.pallas import tpu as pltpu
from jax.experimental.pallas import tpu_sc as plsc

_VOCAB = 100000
_EMBED_DIM = 64
_N_TOKENS = 10
_BATCH = 32
_SEQ = 2048

_CHUNK = 128
_N_CHUNKS = _SEQ // _CHUNK
_GROUP = 4
_N_GROUPS = _N_CHUNKS // _GROUP


def _build_sc_kernel():
    mesh = plsc.VectorSubcoreMesh(core_axis_name="c", subcore_axis_name="s")

    @functools.partial(
        pl.kernel,
        mesh=mesh,
        compiler_params=pltpu.CompilerParams(use_tc_tiling_on_sc=False),
        out_type=jax.ShapeDtypeStruct((_BATCH * _SEQ, _EMBED_DIM), jnp.float32),
        scratch_types=[
            pltpu.VMEM((_N_CHUNKS, _CHUNK), jnp.int32),
            pltpu.VMEM((_GROUP * _CHUNK, _EMBED_DIM), jnp.float32),
            pltpu.VMEM((_GROUP * _CHUNK, _EMBED_DIM), jnp.float32),
            pltpu.VMEM((_N_TOKENS, _EMBED_DIM), jnp.float32),
            pltpu.SemaphoreType.DMA,
            pltpu.SemaphoreType.DMA,
            pltpu.SemaphoreType.DMA,
            pltpu.SemaphoreType.DMA,
        ],
    )
    def k(tok_hbm, table_hbm, learned_hbm, out_hbm,
          idx_v, rows0, rows1, learned_v, gsem0, gsem1, wsem0, wsem1):
        wid = lax.axis_index("s") * 2 + lax.axis_index("c")
        base = wid * _SEQ

        pltpu.sync_copy(tok_hbm.at[wid], idx_v)
        pltpu.sync_copy(learned_hbm, learned_v)

        bufs = (rows0, rows1)
        gsems = (gsem0, gsem1)
        wsems = (wsem0, wsem1)
        gcopies = [[None] * _GROUP, [None] * _GROUP]
        wcopy = [None, None]

        def fire(g, p):
            for t in range(_GROUP):
                gcopies[p][t] = pltpu.async_copy(
                    table_hbm.at[idx_v.at[g * _GROUP + t]],
                    bufs[p].at[pl.ds(t * _CHUNK, _CHUNK)],
                    gsems[p],
                )

        fire(0, 0)
        for g in range(_N_GROUPS):
            p = g % 2
            for t in range(_GROUP):
                gcopies[p][t].wait()
            if g + 1 < _N_GROUPS:
                if wcopy[1 - p] is not None:
                    wcopy[1 - p].wait()
                    wcopy[1 - p] = None
                fire(g + 1, 1 - p)
            if g == 0:
                for r in range(_N_TOKENS):
                    for c in range(_EMBED_DIM // 16):
                        bufs[p][r, pl.ds(c * 16, 16)] = learned_v[r, pl.ds(c * 16, 16)]
            wcopy[p] = pltpu.async_copy(
                bufs[p],
                out_hbm.at[pl.ds(base + g * _GROUP * _CHUNK, _GROUP * _CHUNK)],
                wsems[p],
            )
        for p in (0, 1):
            if wcopy[p] is not None:
                wcopy[p].wait()

    return k


_sc_kernel = _build_sc_kernel()


@jax.jit
def kernel(tokens, wte_weight, learned_embedding):
    tok = tokens.astype(jnp.int32).reshape(_BATCH, _N_CHUNKS, _CHUNK)
    out = _sc_kernel(tok, wte_weight, learned_embedding)
    return out.reshape(_BATCH, _SEQ, _EMBED_DIM)
